# baseline (speedup 1.0000x reference)
import functools

import jax
import jax.numpy as jnp
from jax import lax
from jax.experimental import pallas as pl
from jax.experimental.pallas import tpu as pltpu
from jax.experimental.pallas import tpu_sc as plsc

_T = 2048
_D = 768
_E = 8
_F = 3072
_C = 512

_NC = 2
_NS = 16
_NW = _NC * _NS
_TPW = _T // _NW

_FB = 512
_NFB = _F // _FB


def _router_body(x_ref, wr_ref, logits_ref, eidx_ref, sidx_ref, gidx_ref,
                 gscale_ref):
    x = x_ref[...]
    wr = wr_ref[...]
    logits = jnp.dot(x, wr, preferred_element_type=jnp.float32)
    logits_ref[...] = logits

    m = jnp.max(logits, axis=-1, keepdims=True)
    iota_e = lax.broadcasted_iota(jnp.int32, (_T, _E), 1)
    eidx = jnp.min(jnp.where(logits == m, iota_e, _E), axis=-1,
                   keepdims=True)
    eidx_ref[...] = eidx

    p = 1.0 / jnp.sum(jnp.exp(logits - m), axis=-1, keepdims=True)

    onehot = (iota_e == eidx).astype(jnp.float32)
    cum = onehot
    k = 1
    while k < _T:
        shifted = jnp.concatenate(
            [jnp.zeros((k, _E), jnp.float32), cum[:_T - k]], axis=0)
        cum = cum + shifted
        k *= 2
    loc = jnp.sum((cum - 1.0) * onehot, axis=-1, keepdims=True)
    kept = loc < float(_C)
    loc_i = loc.astype(jnp.int32)
    slot = eidx * _C + loc_i

    e0 = jnp.broadcast_to(eidx[0:1, :], (_T, 1))
    sidx_ref[...] = jnp.where(kept, slot, _E * _C)
    gidx_ref[...] = jnp.where(kept, slot, e0 * _C)
    gscale_ref[...] = jnp.where(kept, p, 0.0)


_router_call = pl.pallas_call(
    _router_body,
    out_shape=(
        jax.ShapeDtypeStruct((_T, _E), jnp.float32),
        jax.ShapeDtypeStruct((_T, 1), jnp.int32),
        jax.ShapeDtypeStruct((_T, 1), jnp.int32),
        jax.ShapeDtypeStruct((_T, 1), jnp.int32),
        jax.ShapeDtypeStruct((_T, 1), jnp.float32),
    ),
)


def _dispatch_body(flat_hbm, sidx_hbm, ebuf_hbm, idx_v, rows_v, sem):
    wid = lax.axis_index("s") * _NC + lax.axis_index("c")
    base = wid * _TPW
    pltpu.sync_copy(sidx_hbm.at[pl.ds(base, _TPW)], idx_v)
    pltpu.sync_copy(flat_hbm.at[pl.ds(base, _TPW)], rows_v)
    pltpu.async_copy(rows_v, ebuf_hbm.at[idx_v], sem).wait()


@functools.cache
def _dispatch_call():
    return functools.partial(
        pl.kernel,
        out_type=jax.ShapeDtypeStruct((_E * _C + 8, _D), jnp.float32),
        mesh=plsc.VectorSubcoreMesh(core_axis_name="c", subcore_axis_name="s"),
        scratch_types=[
            pltpu.VMEM((_TPW,), jnp.int32),
            pltpu.VMEM((_TPW, _D), jnp.float32),
            pltpu.SemaphoreType.DMA,
        ],
    )(_dispatch_body)


def _ffn_body(x_ref, w1_ref, w2_ref, y_ref, acc_ref):
    fb = pl.program_id(1)
    x = x_ref[...]
    h = jnp.maximum(
        lax.dot_general(x, w1_ref[0], (((1,), (0,)), ((), ())),
                        preferred_element_type=jnp.float32), 0.0)
    contrib = lax.dot_general(h, w2_ref[0], (((1,), (0,)), ((), ())),
                              preferred_element_type=jnp.float32)

    @pl.when(fb == 0)
    def _():
        acc_ref[...] = contrib

    @pl.when(fb > 0)
    def _():
        acc_ref[...] += contrib

    @pl.when(fb == _NFB - 1)
    def _():
        y_ref[...] = acc_ref[...]


_ffn_call = pl.pallas_call(
    _ffn_body,
    grid=(_E, _NFB),
    in_specs=[
        pl.BlockSpec((_C, _D), lambda e, fb: (e, 0)),
        pl.BlockSpec((1, _D, _FB), lambda e, fb: (e, 0, fb)),
        pl.BlockSpec((1, _FB, _D), lambda e, fb: (e, fb, 0)),
    ],
    out_specs=pl.BlockSpec((_C, _D), lambda e, fb: (e, 0)),
    out_shape=jax.ShapeDtypeStruct((_E * _C, _D), jnp.float32),
    scratch_shapes=[pltpu.VMEM((_C, _D), jnp.float32)],
    compiler_params=pltpu.CompilerParams(
        dimension_semantics=("arbitrary", "arbitrary")),
)


def _combine_body(y_hbm, gidx_hbm, ygath_hbm, idx_v, rows_v, sem):
    wid = lax.axis_index("s") * _NC + lax.axis_index("c")
    base = wid * _TPW
    pltpu.sync_copy(gidx_hbm.at[pl.ds(base, _TPW)], idx_v)
    pltpu.async_copy(y_hbm.at[idx_v], rows_v, sem).wait()
    pltpu.sync_copy(rows_v, ygath_hbm.at[pl.ds(base, _TPW)])


@functools.cache
def _combine_call():
    return functools.partial(
        pl.kernel,
        out_type=jax.ShapeDtypeStruct((_T, _D), jnp.float32),
        mesh=plsc.VectorSubcoreMesh(core_axis_name="c", subcore_axis_name="s"),
        scratch_types=[
            pltpu.VMEM((_TPW,), jnp.int32),
            pltpu.VMEM((_TPW, _D), jnp.float32),
            pltpu.SemaphoreType.DMA,
        ],
    )(_combine_body)


def _scale_body(y_ref, s_ref, o_ref):
    o_ref[...] = y_ref[...] * s_ref[...]


_scale_call = pl.pallas_call(
    _scale_body,
    out_shape=jax.ShapeDtypeStruct((_T, _D), jnp.float32),
)


def kernel(hidden_states, w_router, w1, w2):
    B, S, D = hidden_states.shape
    flat = hidden_states.reshape(B * S, D)

    logits, eidx, sidx, gidx, gscale = _router_call(flat, w_router)
    ebuf = _dispatch_call()(flat, sidx.reshape(_T))
    y = _ffn_call(ebuf, w1, w2)
    ygath = _combine_call()(y, gidx.reshape(_T))
    out = _scale_call(ygath, gscale)

    return (out.reshape(B, S, D),
            (logits.reshape(B, S, _E), eidx.reshape(B, S)))


# speedup vs baseline: 1.0930x; 1.0930x over previous
import functools

import jax
import jax.numpy as jnp
from jax import lax
from jax.experimental import pallas as pl
from jax.experimental.pallas import tpu as pltpu
from jax.experimental.pallas import tpu_sc as plsc

_T = 2048
_D = 768
_E = 8
_F = 3072
_C = 512

_NC = 2
_NS = 16
_NW = _NC * _NS
_TPW = _T // _NW

_FB = 768
_NFB = _F // _FB


def _router_body(x_ref, wr_ref, logits_ref, eidx_ref, sidx_ref, gidx_ref,
                 gscale_ref):
    x = x_ref[...]
    wr = wr_ref[...]
    logits = jnp.dot(x, wr, preferred_element_type=jnp.float32)
    logits_ref[...] = logits

    m = jnp.max(logits, axis=-1, keepdims=True)
    iota_e = lax.broadcasted_iota(jnp.int32, (_T, _E), 1)
    eidx = jnp.min(jnp.where(logits == m, iota_e, _E), axis=-1,
                   keepdims=True)
    eidx_ref[...] = eidx

    p = 1.0 / jnp.sum(jnp.exp(logits - m), axis=-1, keepdims=True)

    onehot = (iota_e == eidx).astype(jnp.float32)
    cum = onehot
    k = 1
    while k < _T:
        shifted = jnp.concatenate(
            [jnp.zeros((k, _E), jnp.float32), cum[:_T - k]], axis=0)
        cum = cum + shifted
        k *= 2
    loc = jnp.sum((cum - 1.0) * onehot, axis=-1, keepdims=True)
    kept = loc < float(_C)
    loc_i = loc.astype(jnp.int32)
    slot = eidx * _C + loc_i

    e0 = jnp.broadcast_to(eidx[0:1, :], (_T, 1))
    sidx_ref[...] = jnp.where(kept, slot, _E * _C)
    gidx_ref[...] = jnp.where(kept, slot, e0 * _C)
    gscale_ref[...] = jnp.where(kept, p, 0.0)


_router_call = pl.pallas_call(
    _router_body,
    out_shape=(
        jax.ShapeDtypeStruct((_T, _E), jnp.float32),
        jax.ShapeDtypeStruct((_T, 1), jnp.int32),
        jax.ShapeDtypeStruct((_T, 1), jnp.int32),
        jax.ShapeDtypeStruct((_T, 1), jnp.int32),
        jax.ShapeDtypeStruct((_T, 1), jnp.float32),
    ),
)


def _dispatch_body(flat_hbm, sidx_hbm, ebuf_hbm, idx_v, rows_v, sem):
    wid = lax.axis_index("s") * _NC + lax.axis_index("c")
    base = wid * _TPW
    pltpu.sync_copy(sidx_hbm.at[pl.ds(base, _TPW)], idx_v)
    pltpu.sync_copy(flat_hbm.at[pl.ds(base, _TPW)], rows_v)
    pltpu.async_copy(rows_v, ebuf_hbm.at[idx_v], sem).wait()


@functools.cache
def _dispatch_call():
    return functools.partial(
        pl.kernel,
        out_type=jax.ShapeDtypeStruct((_E * _C + 8, _D), jnp.float32),
        mesh=plsc.VectorSubcoreMesh(core_axis_name="c", subcore_axis_name="s"),
        scratch_types=[
            pltpu.VMEM((_TPW,), jnp.int32),
            pltpu.VMEM((_TPW, _D), jnp.float32),
            pltpu.SemaphoreType.DMA,
        ],
    )(_dispatch_body)


def _ffn_body(x_ref, w1_ref, w2_ref, y_ref, acc_ref):
    fb = pl.program_id(1)
    x = x_ref[...].astype(jnp.bfloat16)
    h = jnp.maximum(
        lax.dot_general(x, w1_ref[0].astype(jnp.bfloat16),
                        (((1,), (0,)), ((), ())),
                        preferred_element_type=jnp.float32), 0.0)
    contrib = lax.dot_general(h.astype(jnp.bfloat16),
                              w2_ref[0].astype(jnp.bfloat16),
                              (((1,), (0,)), ((), ())),
                              preferred_element_type=jnp.float32)

    @pl.when(fb == 0)
    def _():
        acc_ref[...] = contrib

    @pl.when(fb > 0)
    def _():
        acc_ref[...] += contrib

    @pl.when(fb == _NFB - 1)
    def _():
        y_ref[...] = acc_ref[...]


_ffn_call = pl.pallas_call(
    _ffn_body,
    grid=(_E, _NFB),
    in_specs=[
        pl.BlockSpec((_C, _D), lambda e, fb: (e, 0)),
        pl.BlockSpec((1, _D, _FB), lambda e, fb: (e, 0, fb)),
        pl.BlockSpec((1, _FB, _D), lambda e, fb: (e, fb, 0)),
    ],
    out_specs=pl.BlockSpec((_C, _D), lambda e, fb: (e, 0)),
    out_shape=jax.ShapeDtypeStruct((_E * _C, _D), jnp.float32),
    scratch_shapes=[pltpu.VMEM((_C, _D), jnp.float32)],
    compiler_params=pltpu.CompilerParams(
        dimension_semantics=("arbitrary", "arbitrary")),
)


def _combine_body(y_hbm, gidx_hbm, ygath_hbm, idx_v, rows_v, sem):
    wid = lax.axis_index("s") * _NC + lax.axis_index("c")
    base = wid * _TPW
    pltpu.sync_copy(gidx_hbm.at[pl.ds(base, _TPW)], idx_v)
    pltpu.async_copy(y_hbm.at[idx_v], rows_v, sem).wait()
    pltpu.sync_copy(rows_v, ygath_hbm.at[pl.ds(base, _TPW)])


@functools.cache
def _combine_call():
    return functools.partial(
        pl.kernel,
        out_type=jax.ShapeDtypeStruct((_T, _D), jnp.float32),
        mesh=plsc.VectorSubcoreMesh(core_axis_name="c", subcore_axis_name="s"),
        scratch_types=[
            pltpu.VMEM((_TPW,), jnp.int32),
            pltpu.VMEM((_TPW, _D), jnp.float32),
            pltpu.SemaphoreType.DMA,
        ],
    )(_combine_body)


def _scale_body(y_ref, s_ref, o_ref):
    o_ref[...] = y_ref[...] * s_ref[...]


_scale_call = pl.pallas_call(
    _scale_body,
    out_shape=jax.ShapeDtypeStruct((_T, _D), jnp.float32),
)


def kernel(hidden_states, w_router, w1, w2):
    B, S, D = hidden_states.shape
    flat = hidden_states.reshape(B * S, D)

    logits, eidx, sidx, gidx, gscale = _router_call(flat, w_router)
    ebuf = _dispatch_call()(flat, sidx.reshape(_T))
    y = _ffn_call(ebuf, w1, w2)
    ygath = _combine_call()(y, gidx.reshape(_T))
    out = _scale_call(ygath, gscale)

    return (out.reshape(B, S, D),
            (logits.reshape(B, S, _E), eidx.reshape(B, S)))


# speedup vs baseline: 1.2055x; 1.1030x over previous
import functools

import jax
import jax.numpy as jnp
from jax import lax
from jax.experimental import pallas as pl
from jax.experimental.pallas import tpu as pltpu
from jax.experimental.pallas import tpu_sc as plsc

_T = 2048
_D = 768
_E = 8
_F = 3072
_C = 512

_NC = 2
_NS = 16
_NW = _NC * _NS
_TPW = _T // _NW

_FB = 1536
_NFB = _F // _FB


def _router_body(x_ref, wr_ref, logits_ref, eidx_ref, sidx_ref, gidx_ref,
                 gscale_ref):
    x = x_ref[...]
    wr = wr_ref[...]
    logits = jnp.dot(x, wr, preferred_element_type=jnp.float32)
    logits_ref[...] = logits

    m = jnp.max(logits, axis=-1, keepdims=True)
    iota_e = lax.broadcasted_iota(jnp.int32, (_T, _E), 1)
    eidx = jnp.min(jnp.where(logits == m, iota_e, _E), axis=-1,
                   keepdims=True)
    eidx_ref[...] = eidx

    p = 1.0 / jnp.sum(jnp.exp(logits - m), axis=-1, keepdims=True)

    onehot = (iota_e == eidx).astype(jnp.float32)
    cum = onehot
    k = 1
    while k < _T:
        shifted = jnp.concatenate(
            [jnp.zeros((k, _E), jnp.float32), cum[:_T - k]], axis=0)
        cum = cum + shifted
        k *= 2
    loc = jnp.sum((cum - 1.0) * onehot, axis=-1, keepdims=True)
    kept = loc < float(_C)
    loc_i = loc.astype(jnp.int32)
    slot = eidx * _C + loc_i

    e0 = jnp.broadcast_to(eidx[0:1, :], (_T, 1))
    sidx_ref[...] = jnp.where(kept, slot, _E * _C)
    gidx_ref[...] = jnp.where(kept, slot, e0 * _C)
    gscale_ref[...] = jnp.where(kept, p, 0.0)


_router_call = pl.pallas_call(
    _router_body,
    out_shape=(
        jax.ShapeDtypeStruct((_T, _E), jnp.float32),
        jax.ShapeDtypeStruct((_T, 1), jnp.int32),
        jax.ShapeDtypeStruct((_T, 1), jnp.int32),
        jax.ShapeDtypeStruct((_T, 1), jnp.int32),
        jax.ShapeDtypeStruct((_T, 1), jnp.float32),
    ),
)


def _dispatch_body(flat_hbm, sidx_hbm, ebuf_hbm, idx_v, rows_v, sem):
    wid = lax.axis_index("s") * _NC + lax.axis_index("c")
    base = wid * _TPW
    pltpu.sync_copy(sidx_hbm.at[pl.ds(base, _TPW)], idx_v)
    pltpu.sync_copy(flat_hbm.at[pl.ds(base, _TPW)], rows_v)
    pltpu.async_copy(rows_v, ebuf_hbm.at[idx_v], sem).wait()


@functools.cache
def _dispatch_call():
    return functools.partial(
        pl.kernel,
        out_type=jax.ShapeDtypeStruct((_E * _C + 8, _D), jnp.float32),
        mesh=plsc.VectorSubcoreMesh(core_axis_name="c", subcore_axis_name="s"),
        scratch_types=[
            pltpu.VMEM((_TPW,), jnp.int32),
            pltpu.VMEM((_TPW, _D), jnp.float32),
            pltpu.SemaphoreType.DMA,
        ],
    )(_dispatch_body)


def _ffn_body(x_ref, w1_ref, w2_ref, y_ref, acc_ref):
    fb = pl.program_id(1)
    x = x_ref[...].astype(jnp.bfloat16)
    h = jnp.maximum(
        lax.dot_general(x, w1_ref[0].astype(jnp.bfloat16),
                        (((1,), (0,)), ((), ())),
                        preferred_element_type=jnp.float32), 0.0)
    contrib = lax.dot_general(h.astype(jnp.bfloat16),
                              w2_ref[0].astype(jnp.bfloat16),
                              (((1,), (0,)), ((), ())),
                              preferred_element_type=jnp.float32)

    @pl.when(fb == 0)
    def _():
        acc_ref[...] = contrib

    @pl.when(fb > 0)
    def _():
        acc_ref[...] += contrib

    @pl.when(fb == _NFB - 1)
    def _():
        y_ref[...] = acc_ref[...]


_ffn_call = pl.pallas_call(
    _ffn_body,
    grid=(_E, _NFB),
    in_specs=[
        pl.BlockSpec((_C, _D), lambda e, fb: (e, 0)),
        pl.BlockSpec((1, _D, _FB), lambda e, fb: (e, 0, fb)),
        pl.BlockSpec((1, _FB, _D), lambda e, fb: (e, fb, 0)),
    ],
    out_specs=pl.BlockSpec((_C, _D), lambda e, fb: (e, 0)),
    out_shape=jax.ShapeDtypeStruct((_E * _C, _D), jnp.float32),
    scratch_shapes=[pltpu.VMEM((_C, _D), jnp.float32)],
    compiler_params=pltpu.CompilerParams(
        dimension_semantics=("arbitrary", "arbitrary")),
)


def _combine_body(y_hbm, gidx_hbm, ygath_hbm, idx_v, rows_v, sem):
    wid = lax.axis_index("s") * _NC + lax.axis_index("c")
    base = wid * _TPW
    pltpu.sync_copy(gidx_hbm.at[pl.ds(base, _TPW)], idx_v)
    pltpu.async_copy(y_hbm.at[idx_v], rows_v, sem).wait()
    pltpu.sync_copy(rows_v, ygath_hbm.at[pl.ds(base, _TPW)])


@functools.cache
def _combine_call():
    return functools.partial(
        pl.kernel,
        out_type=jax.ShapeDtypeStruct((_T, _D), jnp.float32),
        mesh=plsc.VectorSubcoreMesh(core_axis_name="c", subcore_axis_name="s"),
        scratch_types=[
            pltpu.VMEM((_TPW,), jnp.int32),
            pltpu.VMEM((_TPW, _D), jnp.float32),
            pltpu.SemaphoreType.DMA,
        ],
    )(_combine_body)


def _scale_body(y_ref, s_ref, o_ref):
    o_ref[...] = y_ref[...] * s_ref[...]


_scale_call = pl.pallas_call(
    _scale_body,
    out_shape=jax.ShapeDtypeStruct((_T, _D), jnp.float32),
)


def kernel(hidden_states, w_router, w1, w2):
    B, S, D = hidden_states.shape
    flat = hidden_states.reshape(B * S, D)

    logits, eidx, sidx, gidx, gscale = _router_call(flat, w_router)
    ebuf = _dispatch_call()(flat, sidx.reshape(_T))
    y = _ffn_call(ebuf, w1, w2)
    ygath = _combine_call()(y, gidx.reshape(_T))
    out = _scale_call(ygath, gscale)

    return (out.reshape(B, S, D),
            (logits.reshape(B, S, _E), eidx.reshape(B, S)))


# speedup vs baseline: 1.2606x; 1.0457x over previous
import functools

import jax
import jax.numpy as jnp
from jax import lax
from jax.experimental import pallas as pl
from jax.experimental.pallas import tpu as pltpu
from jax.experimental.pallas import tpu_sc as plsc

_T = 2048
_D = 768
_E = 8
_F = 3072
_C = 512

_NC = 2
_NS = 16
_NW = _NC * _NS
_TPW = _T // _NW

_FB = 3072
_NFB = _F // _FB


def _router_body(x_ref, wr_ref, logits_ref, eidx_ref, sidx_ref, gidx_ref,
                 gscale_ref):
    x = x_ref[...]
    wr = wr_ref[...]
    logits = jnp.dot(x, wr, preferred_element_type=jnp.float32)
    logits_ref[...] = logits

    m = jnp.max(logits, axis=-1, keepdims=True)
    iota_e = lax.broadcasted_iota(jnp.int32, (_T, _E), 1)
    eidx = jnp.min(jnp.where(logits == m, iota_e, _E), axis=-1,
                   keepdims=True)
    eidx_ref[...] = eidx

    p = 1.0 / jnp.sum(jnp.exp(logits - m), axis=-1, keepdims=True)

    onehot = (iota_e == eidx).astype(jnp.float32)
    cum = onehot
    k = 1
    while k < _T:
        shifted = jnp.concatenate(
            [jnp.zeros((k, _E), jnp.float32), cum[:_T - k]], axis=0)
        cum = cum + shifted
        k *= 2
    loc = jnp.sum((cum - 1.0) * onehot, axis=-1, keepdims=True)
    kept = loc < float(_C)
    loc_i = loc.astype(jnp.int32)
    slot = eidx * _C + loc_i

    e0 = jnp.broadcast_to(eidx[0:1, :], (_T, 1))
    sidx_ref[...] = jnp.where(kept, slot, _E * _C)
    gidx_ref[...] = jnp.where(kept, slot, e0 * _C)
    gscale_ref[...] = jnp.where(kept, p, 0.0)


_router_call = pl.pallas_call(
    _router_body,
    out_shape=(
        jax.ShapeDtypeStruct((_T, _E), jnp.float32),
        jax.ShapeDtypeStruct((_T, 1), jnp.int32),
        jax.ShapeDtypeStruct((_T, 1), jnp.int32),
        jax.ShapeDtypeStruct((_T, 1), jnp.int32),
        jax.ShapeDtypeStruct((_T, 1), jnp.float32),
    ),
)


def _dispatch_body(flat_hbm, sidx_hbm, ebuf_hbm, idx_v, rows_v, sem):
    wid = lax.axis_index("s") * _NC + lax.axis_index("c")
    base = wid * _TPW
    pltpu.sync_copy(sidx_hbm.at[pl.ds(base, _TPW)], idx_v)
    pltpu.sync_copy(flat_hbm.at[pl.ds(base, _TPW)], rows_v)
    pltpu.async_copy(rows_v, ebuf_hbm.at[idx_v], sem).wait()


@functools.cache
def _dispatch_call():
    return functools.partial(
        pl.kernel,
        out_type=jax.ShapeDtypeStruct((_E * _C + 8, _D), jnp.float32),
        mesh=plsc.VectorSubcoreMesh(core_axis_name="c", subcore_axis_name="s"),
        scratch_types=[
            pltpu.VMEM((_TPW,), jnp.int32),
            pltpu.VMEM((_TPW, _D), jnp.float32),
            pltpu.SemaphoreType.DMA,
        ],
    )(_dispatch_body)


def _ffn_body(x_ref, w1_ref, w2_ref, y_ref, acc_ref):
    fb = pl.program_id(1)
    x = x_ref[...].astype(jnp.bfloat16)
    h = jnp.maximum(
        lax.dot_general(x, w1_ref[0].astype(jnp.bfloat16),
                        (((1,), (0,)), ((), ())),
                        preferred_element_type=jnp.float32), 0.0)
    contrib = lax.dot_general(h.astype(jnp.bfloat16),
                              w2_ref[0].astype(jnp.bfloat16),
                              (((1,), (0,)), ((), ())),
                              preferred_element_type=jnp.float32)

    @pl.when(fb == 0)
    def _():
        acc_ref[...] = contrib

    @pl.when(fb > 0)
    def _():
        acc_ref[...] += contrib

    @pl.when(fb == _NFB - 1)
    def _():
        y_ref[...] = acc_ref[...]


_ffn_call = pl.pallas_call(
    _ffn_body,
    grid=(_E, _NFB),
    in_specs=[
        pl.BlockSpec((_C, _D), lambda e, fb: (e, 0)),
        pl.BlockSpec((1, _D, _FB), lambda e, fb: (e, 0, fb)),
        pl.BlockSpec((1, _FB, _D), lambda e, fb: (e, fb, 0)),
    ],
    out_specs=pl.BlockSpec((_C, _D), lambda e, fb: (e, 0)),
    out_shape=jax.ShapeDtypeStruct((_E * _C, _D), jnp.float32),
    scratch_shapes=[pltpu.VMEM((_C, _D), jnp.float32)],
    compiler_params=pltpu.CompilerParams(
        dimension_semantics=("arbitrary", "arbitrary")),
)


def _combine_body(y_hbm, gidx_hbm, ygath_hbm, idx_v, rows_v, sem):
    wid = lax.axis_index("s") * _NC + lax.axis_index("c")
    base = wid * _TPW
    pltpu.sync_copy(gidx_hbm.at[pl.ds(base, _TPW)], idx_v)
    pltpu.async_copy(y_hbm.at[idx_v], rows_v, sem).wait()
    pltpu.sync_copy(rows_v, ygath_hbm.at[pl.ds(base, _TPW)])


@functools.cache
def _combine_call():
    return functools.partial(
        pl.kernel,
        out_type=jax.ShapeDtypeStruct((_T, _D), jnp.float32),
        mesh=plsc.VectorSubcoreMesh(core_axis_name="c", subcore_axis_name="s"),
        scratch_types=[
            pltpu.VMEM((_TPW,), jnp.int32),
            pltpu.VMEM((_TPW, _D), jnp.float32),
            pltpu.SemaphoreType.DMA,
        ],
    )(_combine_body)


def _scale_body(y_ref, s_ref, o_ref):
    o_ref[...] = y_ref[...] * s_ref[...]


_scale_call = pl.pallas_call(
    _scale_body,
    out_shape=jax.ShapeDtypeStruct((_T, _D), jnp.float32),
)


def kernel(hidden_states, w_router, w1, w2):
    B, S, D = hidden_states.shape
    flat = hidden_states.reshape(B * S, D)

    logits, eidx, sidx, gidx, gscale = _router_call(flat, w_router)
    ebuf = _dispatch_call()(flat, sidx.reshape(_T))
    y = _ffn_call(ebuf, w1, w2)
    ygath = _combine_call()(y, gidx.reshape(_T))
    out = _scale_call(ygath, gscale)

    return (out.reshape(B, S, D),
            (logits.reshape(B, S, _E), eidx.reshape(B, S)))


# speedup vs baseline: 1.2647x; 1.0032x over previous
import functools

import jax
import jax.numpy as jnp
from jax import lax
from jax.experimental import pallas as pl
from jax.experimental.pallas import tpu as pltpu
from jax.experimental.pallas import tpu_sc as plsc

_T = 2048
_D = 768
_E = 8
_F = 3072
_C = 512

_NC = 2
_NS = 16
_NW = _NC * _NS
_TPW = _T // _NW

_FB = 3072
_NFB = _F // _FB


def _router_body(x_ref, wr_ref, logits_ref, eidx_ref, sidx_ref, gidx_ref,
                 gscale_ref):
    x = x_ref[...]
    wr = wr_ref[...]
    logits = jnp.dot(x, wr, preferred_element_type=jnp.float32)
    logits_ref[...] = logits

    m = jnp.max(logits, axis=-1, keepdims=True)
    iota_e = lax.broadcasted_iota(jnp.int32, (_T, _E), 1)
    eidx = jnp.min(jnp.where(logits == m, iota_e, _E), axis=-1,
                   keepdims=True)
    eidx_ref[...] = eidx

    p = 1.0 / jnp.sum(jnp.exp(logits - m), axis=-1, keepdims=True)

    onehot = (iota_e == eidx).astype(jnp.float32)
    cum = onehot
    k = 1
    while k < _T:
        shifted = jnp.concatenate(
            [jnp.zeros((k, _E), jnp.float32), cum[:_T - k]], axis=0)
        cum = cum + shifted
        k *= 2
    loc = jnp.sum((cum - 1.0) * onehot, axis=-1, keepdims=True)
    kept = loc < float(_C)
    loc_i = loc.astype(jnp.int32)
    slot = eidx * _C + loc_i

    e0 = jnp.broadcast_to(eidx[0:1, :], (_T, 1))
    sidx_ref[...] = jnp.where(kept, slot, _E * _C)
    gidx_ref[...] = jnp.where(kept, slot, e0 * _C)
    gscale_ref[...] = jnp.where(kept, p, 0.0)


_router_call = pl.pallas_call(
    _router_body,
    out_shape=(
        jax.ShapeDtypeStruct((_T, _E), jnp.float32),
        jax.ShapeDtypeStruct((_T, 1), jnp.int32),
        jax.ShapeDtypeStruct((_T, 1), jnp.int32),
        jax.ShapeDtypeStruct((_T, 1), jnp.int32),
        jax.ShapeDtypeStruct((_T, 1), jnp.float32),
    ),
)


def _dispatch_body(flat_hbm, sidx_hbm, ebuf_hbm, idx_v, rows_v, sem):
    wid = lax.axis_index("s") * _NC + lax.axis_index("c")
    base = wid * _TPW
    pltpu.sync_copy(sidx_hbm.at[pl.ds(base, _TPW)], idx_v)
    pltpu.sync_copy(flat_hbm.at[pl.ds(base, _TPW)], rows_v)
    pltpu.async_copy(rows_v, ebuf_hbm.at[idx_v], sem).wait()


@functools.cache
def _dispatch_call():
    return functools.partial(
        pl.kernel,
        out_type=jax.ShapeDtypeStruct((_E * _C + 8, _D), jnp.float32),
        mesh=plsc.VectorSubcoreMesh(core_axis_name="c", subcore_axis_name="s"),
        scratch_types=[
            pltpu.VMEM((_TPW,), jnp.int32),
            pltpu.VMEM((_TPW, _D), jnp.float32),
            pltpu.SemaphoreType.DMA,
        ],
    )(_dispatch_body)


def _ffn_body(x_ref, w1_ref, w2_ref, y_ref):
    x = x_ref[...].astype(jnp.bfloat16)
    h = jnp.maximum(
        lax.dot_general(x, w1_ref[0].astype(jnp.bfloat16),
                        (((1,), (0,)), ((), ())),
                        preferred_element_type=jnp.float32), 0.0)
    y_ref[...] = lax.dot_general(h.astype(jnp.bfloat16),
                                 w2_ref[0].astype(jnp.bfloat16),
                                 (((1,), (0,)), ((), ())),
                                 preferred_element_type=jnp.float32)


_ffn_call = pl.pallas_call(
    _ffn_body,
    grid=(_E,),
    in_specs=[
        pl.BlockSpec((_C, _D), lambda e: (e, 0)),
        pl.BlockSpec((1, _D, _F), lambda e: (e, 0, 0)),
        pl.BlockSpec((1, _F, _D), lambda e: (e, 0, 0)),
    ],
    out_specs=pl.BlockSpec((_C, _D), lambda e: (e, 0)),
    out_shape=jax.ShapeDtypeStruct((_E * _C, _D), jnp.float32),
    compiler_params=pltpu.CompilerParams(
        dimension_semantics=("arbitrary",)),
)


def _combine_body(y_hbm, gidx_hbm, ygath_hbm, idx_v, rows_v, sem):
    wid = lax.axis_index("s") * _NC + lax.axis_index("c")
    base = wid * _TPW
    pltpu.sync_copy(gidx_hbm.at[pl.ds(base, _TPW)], idx_v)
    pltpu.async_copy(y_hbm.at[idx_v], rows_v, sem).wait()
    pltpu.sync_copy(rows_v, ygath_hbm.at[pl.ds(base, _TPW)])


@functools.cache
def _combine_call():
    return functools.partial(
        pl.kernel,
        out_type=jax.ShapeDtypeStruct((_T, _D), jnp.float32),
        mesh=plsc.VectorSubcoreMesh(core_axis_name="c", subcore_axis_name="s"),
        scratch_types=[
            pltpu.VMEM((_TPW,), jnp.int32),
            pltpu.VMEM((_TPW, _D), jnp.float32),
            pltpu.SemaphoreType.DMA,
        ],
    )(_combine_body)


def _scale_body(y_ref, s_ref, o_ref):
    o_ref[...] = y_ref[...] * s_ref[...]


_scale_call = pl.pallas_call(
    _scale_body,
    out_shape=jax.ShapeDtypeStruct((_T, _D), jnp.float32),
)


def kernel(hidden_states, w_router, w1, w2):
    B, S, D = hidden_states.shape
    flat = hidden_states.reshape(B * S, D)

    logits, eidx, sidx, gidx, gscale = _router_call(flat, w_router)
    ebuf = _dispatch_call()(flat, sidx.reshape(_T))
    y = _ffn_call(ebuf, w1, w2)
    ygath = _combine_call()(y, gidx.reshape(_T))
    out = _scale_call(ygath, gscale)

    return (out.reshape(B, S, D),
            (logits.reshape(B, S, _E), eidx.reshape(B, S)))


# speedup vs baseline: 1.2914x; 1.0212x over previous
import functools

import jax
import jax.numpy as jnp
from jax import lax
from jax.experimental import pallas as pl
from jax.experimental.pallas import tpu as pltpu
from jax.experimental.pallas import tpu_sc as plsc

_T = 2048
_D = 768
_E = 8
_F = 3072
_C = 512

_NC = 2
_NS = 16
_NW = _NC * _NS
_TPW = _T // _NW

_FB = 3072
_NFB = _F // _FB


def _router_body(x_ref, wr_ref, logits_ref, eidx_ref, sidx_ref, gidx_ref,
                 gscale_ref):
    x = x_ref[...]
    wr = wr_ref[...]
    logits = jnp.dot(x, wr, preferred_element_type=jnp.float32)
    logits_ref[...] = logits

    m = jnp.max(logits, axis=-1, keepdims=True)
    iota_e = lax.broadcasted_iota(jnp.int32, (_T, _E), 1)
    eidx = jnp.min(jnp.where(logits == m, iota_e, _E), axis=-1,
                   keepdims=True)
    eidx_ref[...] = eidx

    p = 1.0 / jnp.sum(jnp.exp(logits - m), axis=-1, keepdims=True)

    onehot = (iota_e == eidx).astype(jnp.float32)
    cum = onehot
    k = 1
    while k < _T:
        shifted = jnp.concatenate(
            [jnp.zeros((k, _E), jnp.float32), cum[:_T - k]], axis=0)
        cum = cum + shifted
        k *= 2
    loc = jnp.sum((cum - 1.0) * onehot, axis=-1, keepdims=True)
    kept = loc < float(_C)
    loc_i = loc.astype(jnp.int32)
    slot = eidx * _C + loc_i

    sidx_ref[...] = jnp.where(kept, slot, _E * _C)
    gidx_ref[...] = jnp.where(kept, slot, _E * _C)
    gscale_ref[...] = jnp.broadcast_to(jnp.where(kept, p, 0.0), (_T, 128))


_router_call = pl.pallas_call(
    _router_body,
    out_shape=(
        jax.ShapeDtypeStruct((_T, _E), jnp.float32),
        jax.ShapeDtypeStruct((_T, 1), jnp.int32),
        jax.ShapeDtypeStruct((_T, 1), jnp.int32),
        jax.ShapeDtypeStruct((_T, 1), jnp.int32),
        jax.ShapeDtypeStruct((_T, 128), jnp.float32),
    ),
)


def _dispatch_body(flat_hbm, sidx_hbm, gs_hbm, ebuf_hbm, sbuf_hbm,
                   idx_v, rows_v, gs_v, sem, sem2):
    wid = lax.axis_index("s") * _NC + lax.axis_index("c")
    base = wid * _TPW
    pltpu.sync_copy(sidx_hbm.at[pl.ds(base, _TPW)], idx_v)
    pltpu.sync_copy(flat_hbm.at[pl.ds(base, _TPW)], rows_v)
    pltpu.sync_copy(gs_hbm.at[pl.ds(base, _TPW)], gs_v)
    cp1 = pltpu.async_copy(rows_v, ebuf_hbm.at[idx_v], sem)
    cp2 = pltpu.async_copy(gs_v, sbuf_hbm.at[idx_v], sem2)
    cp1.wait()
    cp2.wait()


@functools.cache
def _dispatch_call():
    return functools.partial(
        pl.kernel,
        out_type=(
            jax.ShapeDtypeStruct((_E * _C + 8, _D), jnp.float32),
            jax.ShapeDtypeStruct((_E * _C + 8, 128), jnp.float32),
        ),
        mesh=plsc.VectorSubcoreMesh(core_axis_name="c", subcore_axis_name="s"),
        scratch_types=[
            pltpu.VMEM((_TPW,), jnp.int32),
            pltpu.VMEM((_TPW, _D), jnp.float32),
            pltpu.VMEM((_TPW, 128), jnp.float32),
            pltpu.SemaphoreType.DMA,
            pltpu.SemaphoreType.DMA,
        ],
    )(_dispatch_body)


def _ffn_body(x_ref, w1_ref, w2_ref, ss_ref, y_ref):
    e = pl.program_id(0)

    @pl.when(e < _E)
    def _():
        x = x_ref[...].astype(jnp.bfloat16)
        h = jnp.maximum(
            lax.dot_general(x, w1_ref[0].astype(jnp.bfloat16),
                            (((1,), (0,)), ((), ())),
                            preferred_element_type=jnp.float32), 0.0)
        y = lax.dot_general(h.astype(jnp.bfloat16),
                            w2_ref[0].astype(jnp.bfloat16),
                            (((1,), (0,)), ((), ())),
                            preferred_element_type=jnp.float32)
        y_ref[...] = y * ss_ref[:, 0:1]

    @pl.when(e == _E)
    def _():
        y_ref[...] = jnp.zeros((_C, _D), jnp.float32)


def _clampe(e):
    return jnp.minimum(e, _E - 1)


_ffn_call = pl.pallas_call(
    _ffn_body,
    grid=(_E + 1,),
    in_specs=[
        pl.BlockSpec((_C, _D), lambda e: (_clampe(e), 0)),
        pl.BlockSpec((1, _D, _F), lambda e: (_clampe(e), 0, 0)),
        pl.BlockSpec((1, _F, _D), lambda e: (_clampe(e), 0, 0)),
        pl.BlockSpec((_C, 128), lambda e: (_clampe(e), 0)),
    ],
    out_specs=pl.BlockSpec((_C, _D), lambda e: (e, 0)),
    out_shape=jax.ShapeDtypeStruct(((_E + 1) * _C, _D), jnp.float32),
    compiler_params=pltpu.CompilerParams(
        dimension_semantics=("arbitrary",)),
)


def _combine_body(y_hbm, gidx_hbm, ygath_hbm, idx_v, rows_v, sem):
    wid = lax.axis_index("s") * _NC + lax.axis_index("c")
    base = wid * _TPW
    pltpu.sync_copy(gidx_hbm.at[pl.ds(base, _TPW)], idx_v)
    pltpu.async_copy(y_hbm.at[idx_v], rows_v, sem).wait()
    pltpu.sync_copy(rows_v, ygath_hbm.at[pl.ds(base, _TPW)])


@functools.cache
def _combine_call():
    return functools.partial(
        pl.kernel,
        out_type=jax.ShapeDtypeStruct((_T, _D), jnp.float32),
        mesh=plsc.VectorSubcoreMesh(core_axis_name="c", subcore_axis_name="s"),
        scratch_types=[
            pltpu.VMEM((_TPW,), jnp.int32),
            pltpu.VMEM((_TPW, _D), jnp.float32),
            pltpu.SemaphoreType.DMA,
        ],
    )(_combine_body)


def kernel(hidden_states, w_router, w1, w2):
    B, S, D = hidden_states.shape
    flat = hidden_states.reshape(B * S, D)

    logits, eidx, sidx, gidx, gscale = _router_call(flat, w_router)
    ebuf, sbuf = _dispatch_call()(flat, sidx.reshape(_T), gscale)
    y = _ffn_call(ebuf, w1, w2, sbuf)
    out = _combine_call()(y, gidx.reshape(_T))

    return (out.reshape(B, S, D),
            (logits.reshape(B, S, _E), eidx.reshape(B, S)))


# speedup vs baseline: 1.3127x; 1.0165x over previous
import functools

import jax
import jax.numpy as jnp
from jax import lax
from jax.experimental import pallas as pl
from jax.experimental.pallas import tpu as pltpu
from jax.experimental.pallas import tpu_sc as plsc

_T = 2048
_D = 768
_E = 8
_F = 3072
_C = 512

_NC = 2
_NS = 16
_NW = _NC * _NS
_TPW = _T // _NW

_FB = 3072
_NFB = _F // _FB


def _router_body(x_ref, wr_ref, logits_ref, eidx_ref, sidx_ref, gidx_ref,
                 gscale_ref):
    x = x_ref[...]
    wr = wr_ref[...]
    logits = jnp.dot(x, wr, preferred_element_type=jnp.float32)
    logits_ref[...] = logits

    m = jnp.max(logits, axis=-1, keepdims=True)
    iota_e = lax.broadcasted_iota(jnp.int32, (_T, _E), 1)
    eidx = jnp.min(jnp.where(logits == m, iota_e, _E), axis=-1,
                   keepdims=True)
    eidx_ref[...] = eidx.reshape(_T)

    p = 1.0 / jnp.sum(jnp.exp(logits - m), axis=-1, keepdims=True)

    onehot = (iota_e == eidx).astype(jnp.float32)
    cum = onehot
    k = 1
    while k < _T:
        shifted = jnp.concatenate(
            [jnp.zeros((k, _E), jnp.float32), cum[:_T - k]], axis=0)
        cum = cum + shifted
        k *= 2
    loc = jnp.sum((cum - 1.0) * onehot, axis=-1, keepdims=True)
    kept = loc < float(_C)
    loc_i = loc.astype(jnp.int32)
    slot = eidx * _C + loc_i

    sidx = jnp.where(kept, slot, _E * _C)
    sidx_ref[...] = sidx.reshape(_T)
    gidx_ref[...] = jnp.where(kept, slot, _E * _C).reshape(_T)
    gscale_ref[...] = jnp.broadcast_to(jnp.where(kept, p, 0.0), (_T, 128))


_router_call = pl.pallas_call(
    _router_body,
    out_shape=(
        jax.ShapeDtypeStruct((_T, _E), jnp.float32),
        jax.ShapeDtypeStruct((_T,), jnp.int32),
        jax.ShapeDtypeStruct((_T,), jnp.int32),
        jax.ShapeDtypeStruct((_T,), jnp.int32),
        jax.ShapeDtypeStruct((_T, 128), jnp.float32),
    ),
)


def _dispatch_body(flat_hbm, sidx_hbm, gs_hbm, ebuf_hbm, sbuf_hbm,
                   idx_v, rows_v, gs_v, sem, sem2):
    wid = lax.axis_index("s") * _NC + lax.axis_index("c")
    base = wid * _TPW
    pltpu.sync_copy(sidx_hbm.at[pl.ds(base, _TPW)], idx_v)
    pltpu.sync_copy(flat_hbm.at[pl.ds(base, _TPW)], rows_v)
    pltpu.sync_copy(gs_hbm.at[pl.ds(base, _TPW)], gs_v)
    cp1 = pltpu.async_copy(rows_v, ebuf_hbm.at[idx_v], sem)
    cp2 = pltpu.async_copy(gs_v, sbuf_hbm.at[idx_v], sem2)
    cp1.wait()
    cp2.wait()


@functools.cache
def _dispatch_call():
    return functools.partial(
        pl.kernel,
        out_type=(
            jax.ShapeDtypeStruct((_E * _C + 8, _D), jnp.float32),
            jax.ShapeDtypeStruct((_E * _C + 8, 128), jnp.float32),
        ),
        mesh=plsc.VectorSubcoreMesh(core_axis_name="c", subcore_axis_name="s"),
        scratch_types=[
            pltpu.VMEM((_TPW,), jnp.int32),
            pltpu.VMEM((_TPW, _D), jnp.float32),
            pltpu.VMEM((_TPW, 128), jnp.float32),
            pltpu.SemaphoreType.DMA,
            pltpu.SemaphoreType.DMA,
        ],
    )(_dispatch_body)


def _ffn_body(x_ref, w1_ref, w2_ref, ss_ref, y_ref):
    e = pl.program_id(0)

    @pl.when(e < _E)
    def _():
        x = x_ref[...].astype(jnp.bfloat16)
        h = jnp.maximum(
            lax.dot_general(x, w1_ref[0].astype(jnp.bfloat16),
                            (((1,), (0,)), ((), ())),
                            preferred_element_type=jnp.float32), 0.0)
        y = lax.dot_general(h.astype(jnp.bfloat16),
                            w2_ref[0].astype(jnp.bfloat16),
                            (((1,), (0,)), ((), ())),
                            preferred_element_type=jnp.float32)
        y_ref[...] = y * ss_ref[:, 0:1]

    @pl.when(e == _E)
    def _():
        y_ref[...] = jnp.zeros((_C, _D), jnp.float32)


def _clampe(e):
    return jnp.minimum(e, _E - 1)


_ffn_call = pl.pallas_call(
    _ffn_body,
    grid=(_E + 1,),
    in_specs=[
        pl.BlockSpec((_C, _D), lambda e: (_clampe(e), 0)),
        pl.BlockSpec((1, _D, _F), lambda e: (_clampe(e), 0, 0)),
        pl.BlockSpec((1, _F, _D), lambda e: (_clampe(e), 0, 0)),
        pl.BlockSpec((_C, 128), lambda e: (_clampe(e), 0)),
    ],
    out_specs=pl.BlockSpec((_C, _D), lambda e: (e, 0)),
    out_shape=jax.ShapeDtypeStruct(((_E + 1) * _C, _D), jnp.float32),
    compiler_params=pltpu.CompilerParams(
        dimension_semantics=("arbitrary",)),
)


def _combine_body(y_hbm, gidx_hbm, ygath_hbm, idx_v, rows_v, sem):
    wid = lax.axis_index("s") * _NC + lax.axis_index("c")
    base = wid * _TPW
    pltpu.sync_copy(gidx_hbm.at[pl.ds(base, _TPW)], idx_v)
    pltpu.async_copy(y_hbm.at[idx_v], rows_v, sem).wait()
    pltpu.sync_copy(rows_v, ygath_hbm.at[pl.ds(base, _TPW)])


@functools.cache
def _combine_call():
    return functools.partial(
        pl.kernel,
        out_type=jax.ShapeDtypeStruct((_T, _D), jnp.float32),
        mesh=plsc.VectorSubcoreMesh(core_axis_name="c", subcore_axis_name="s"),
        scratch_types=[
            pltpu.VMEM((_TPW,), jnp.int32),
            pltpu.VMEM((_TPW, _D), jnp.float32),
            pltpu.SemaphoreType.DMA,
        ],
    )(_combine_body)


def kernel(hidden_states, w_router, w1, w2):
    B, S, D = hidden_states.shape
    flat = hidden_states.reshape(B * S, D)

    logits, eidx, sidx, gidx, gscale = _router_call(flat, w_router)
    ebuf, sbuf = _dispatch_call()(flat, sidx, gscale)
    y = _ffn_call(ebuf, w1, w2, sbuf)
    out = _combine_call()(y, gidx)

    return (out.reshape(B, S, D),
            (logits.reshape(B, S, _E), eidx.reshape(B, S)))


# speedup vs baseline: 1.3582x; 1.0346x over previous
import functools

import jax
import jax.numpy as jnp
from jax import lax
from jax.experimental import pallas as pl
from jax.experimental.pallas import tpu as pltpu
from jax.experimental.pallas import tpu_sc as plsc

_T = 2048
_D = 768
_E = 8
_F = 3072
_C = 512

_NC = 2
_NS = 16
_NW = _NC * _NS
_TPW = _T // _NW

_FB = 3072
_NFB = _F // _FB


def _router_body(x_ref, wr_ref, lt_ref, eidx_ref, sidx_ref, gidx_ref,
                 gscale_ref):
    x = x_ref[...]
    wr = wr_ref[...]
    logits = jnp.dot(x, wr, preferred_element_type=jnp.float32)
    lt = logits.T
    lt_ref[...] = lt

    m = jnp.max(lt, axis=0, keepdims=True)
    iota_e = lax.broadcasted_iota(jnp.int32, (_E, _T), 0)
    eidx = jnp.min(jnp.where(lt == m, iota_e, _E), axis=0,
                   keepdims=True)
    eidx_ref[...] = eidx.reshape(_T)

    p = 1.0 / jnp.sum(jnp.exp(lt - m), axis=0, keepdims=True)

    onehot = (iota_e == eidx).astype(jnp.float32)
    cum = onehot
    k = 1
    while k < _T:
        shifted = jnp.concatenate(
            [jnp.zeros((_E, k), jnp.float32), cum[:, :_T - k]], axis=1)
        cum = cum + shifted
        k *= 2
    loc = jnp.sum((cum - 1.0) * onehot, axis=0, keepdims=True)
    kept = loc < float(_C)
    loc_i = loc.astype(jnp.int32)
    slot = eidx * _C + loc_i

    sidx_ref[...] = jnp.where(kept, slot, _E * _C).reshape(_T)
    gidx_ref[...] = jnp.where(kept, slot, _E * _C).reshape(_T)
    gsc = jnp.where(kept, p, 0.0).reshape(_T, 1)
    gscale_ref[...] = jnp.broadcast_to(gsc, (_T, 128))


_router_call = pl.pallas_call(
    _router_body,
    out_shape=(
        jax.ShapeDtypeStruct((_E, _T), jnp.float32),
        jax.ShapeDtypeStruct((_T,), jnp.int32),
        jax.ShapeDtypeStruct((_T,), jnp.int32),
        jax.ShapeDtypeStruct((_T,), jnp.int32),
        jax.ShapeDtypeStruct((_T, 128), jnp.float32),
    ),
)


def _dispatch_body(flat_hbm, sidx_hbm, gs_hbm, ebuf_hbm, sbuf_hbm,
                   idx_v, rows_v, gs_v, sem, sem2):
    wid = lax.axis_index("s") * _NC + lax.axis_index("c")
    base = wid * _TPW
    pltpu.sync_copy(sidx_hbm.at[pl.ds(base, _TPW)], idx_v)
    pltpu.sync_copy(flat_hbm.at[pl.ds(base, _TPW)], rows_v)
    pltpu.sync_copy(gs_hbm.at[pl.ds(base, _TPW)], gs_v)
    cp1 = pltpu.async_copy(rows_v, ebuf_hbm.at[idx_v], sem)
    cp2 = pltpu.async_copy(gs_v, sbuf_hbm.at[idx_v], sem2)
    cp1.wait()
    cp2.wait()


@functools.cache
def _dispatch_call():
    return functools.partial(
        pl.kernel,
        out_type=(
            jax.ShapeDtypeStruct((_E * _C + 8, _D), jnp.float32),
            jax.ShapeDtypeStruct((_E * _C + 8, 128), jnp.float32),
        ),
        mesh=plsc.VectorSubcoreMesh(core_axis_name="c", subcore_axis_name="s"),
        scratch_types=[
            pltpu.VMEM((_TPW,), jnp.int32),
            pltpu.VMEM((_TPW, _D), jnp.float32),
            pltpu.VMEM((_TPW, 128), jnp.float32),
            pltpu.SemaphoreType.DMA,
            pltpu.SemaphoreType.DMA,
        ],
    )(_dispatch_body)


def _ffn_body(x_ref, w1_ref, w2_ref, ss_ref, y_ref):
    e = pl.program_id(0)

    @pl.when(e < _E)
    def _():
        x = x_ref[...].astype(jnp.bfloat16)
        h = jnp.maximum(
            lax.dot_general(x, w1_ref[0].astype(jnp.bfloat16),
                            (((1,), (0,)), ((), ())),
                            preferred_element_type=jnp.float32), 0.0)
        y = lax.dot_general(h.astype(jnp.bfloat16),
                            w2_ref[0].astype(jnp.bfloat16),
                            (((1,), (0,)), ((), ())),
                            preferred_element_type=jnp.float32)
        y_ref[...] = y * ss_ref[:, 0:1]

    @pl.when(e == _E)
    def _():
        y_ref[...] = jnp.zeros((_C, _D), jnp.float32)


def _clampe(e):
    return jnp.minimum(e, _E - 1)


_ffn_call = pl.pallas_call(
    _ffn_body,
    grid=(_E + 1,),
    in_specs=[
        pl.BlockSpec((_C, _D), lambda e: (_clampe(e), 0)),
        pl.BlockSpec((1, _D, _F), lambda e: (_clampe(e), 0, 0)),
        pl.BlockSpec((1, _F, _D), lambda e: (_clampe(e), 0, 0)),
        pl.BlockSpec((_C, 128), lambda e: (_clampe(e), 0)),
    ],
    out_specs=pl.BlockSpec((_C, _D), lambda e: (e, 0)),
    out_shape=jax.ShapeDtypeStruct(((_E + 1) * _C, _D), jnp.float32),
    compiler_params=pltpu.CompilerParams(
        dimension_semantics=("arbitrary",)),
)


def _combine_body(y_hbm, gidx_hbm, ygath_hbm, idx_v, rows_v, sem):
    wid = lax.axis_index("s") * _NC + lax.axis_index("c")
    base = wid * _TPW
    pltpu.sync_copy(gidx_hbm.at[pl.ds(base, _TPW)], idx_v)
    pltpu.async_copy(y_hbm.at[idx_v], rows_v, sem).wait()
    pltpu.sync_copy(rows_v, ygath_hbm.at[pl.ds(base, _TPW)])


@functools.cache
def _combine_call():
    return functools.partial(
        pl.kernel,
        out_type=jax.ShapeDtypeStruct((_T, _D), jnp.float32),
        mesh=plsc.VectorSubcoreMesh(core_axis_name="c", subcore_axis_name="s"),
        scratch_types=[
            pltpu.VMEM((_TPW,), jnp.int32),
            pltpu.VMEM((_TPW, _D), jnp.float32),
            pltpu.SemaphoreType.DMA,
        ],
    )(_combine_body)


def kernel(hidden_states, w_router, w1, w2):
    B, S, D = hidden_states.shape
    flat = hidden_states.reshape(B * S, D)

    lt, eidx, sidx, gidx, gscale = _router_call(flat, w_router)
    ebuf, sbuf = _dispatch_call()(flat, sidx, gscale)
    y = _ffn_call(ebuf, w1, w2, sbuf)
    out = _combine_call()(y, gidx)

    return (out.reshape(B, S, D),
            (lt.T.reshape(B, S, _E), eidx.reshape(B, S)))


# speedup vs baseline: 1.3705x; 1.0091x over previous
import functools

import jax
import jax.numpy as jnp
from jax import lax
from jax.experimental import pallas as pl
from jax.experimental.pallas import tpu as pltpu
from jax.experimental.pallas import tpu_sc as plsc

_T = 2048
_D = 768
_E = 8
_F = 3072
_C = 512

_NC = 2
_NS = 16
_NW = _NC * _NS
_TPW = _T // _NW

_FB = 3072
_NFB = _F // _FB


def _router_body(x_ref, wrt_ref, lt_ref, eidx_ref, sidx_ref,
                 gidx_ref, gscale_ref):
    x = x_ref[...]
    wrt = wrt_ref[...]
    logits = lax.dot_general(x, wrt, (((1,), (1,)), ((), ())),
                             preferred_element_type=jnp.float32)
    lt = logits.T
    lt_ref[...] = lt

    m = jnp.max(lt, axis=0, keepdims=True)
    iota_e = lax.broadcasted_iota(jnp.int32, (_E, _T), 0)
    eidx = jnp.min(jnp.where(lt == m, iota_e, _E), axis=0,
                   keepdims=True)
    eidx_ref[...] = eidx.reshape(_T)

    p = 1.0 / jnp.sum(jnp.exp(lt - m), axis=0, keepdims=True)

    onehot = (iota_e == eidx).astype(jnp.float32)
    cum = onehot
    k = 1
    while k < _T:
        shifted = jnp.concatenate(
            [jnp.zeros((_E, k), jnp.float32), cum[:, :_T - k]], axis=1)
        cum = cum + shifted
        k *= 2
    loc = jnp.sum((cum - 1.0) * onehot, axis=0, keepdims=True)
    kept = loc < float(_C)
    loc_i = loc.astype(jnp.int32)
    slot = eidx * _C + loc_i

    sidx_ref[...] = jnp.where(kept, slot, _E * _C).reshape(_T)
    gidx_ref[...] = jnp.where(kept, slot, _E * _C).reshape(_T)
    gsc = jnp.where(kept, p, 0.0).reshape(_T, 1)
    gscale_ref[...] = jnp.broadcast_to(gsc, (_T, 128))


_router_call = pl.pallas_call(
    _router_body,
    out_shape=(
        jax.ShapeDtypeStruct((_E, _T), jnp.float32),
        jax.ShapeDtypeStruct((_T,), jnp.int32),
        jax.ShapeDtypeStruct((_T,), jnp.int32),
        jax.ShapeDtypeStruct((_T,), jnp.int32),
        jax.ShapeDtypeStruct((_T, 128), jnp.float32),
    ),
)


def _dispatch_body(flat_hbm, sidx_hbm, gs_hbm, ebuf_hbm, sbuf_hbm,
                   idx_v, rows_v, gs_v, sem, sem2):
    wid = lax.axis_index("s") * _NC + lax.axis_index("c")
    base = wid * _TPW
    pltpu.sync_copy(sidx_hbm.at[pl.ds(base, _TPW)], idx_v)
    pltpu.sync_copy(flat_hbm.at[pl.ds(base, _TPW)], rows_v)
    pltpu.sync_copy(gs_hbm.at[pl.ds(base, _TPW)], gs_v)
    cp1 = pltpu.async_copy(rows_v, ebuf_hbm.at[idx_v], sem)
    cp2 = pltpu.async_copy(gs_v, sbuf_hbm.at[idx_v], sem2)
    cp1.wait()
    cp2.wait()


@functools.cache
def _dispatch_call():
    return functools.partial(
        pl.kernel,
        out_type=(
            jax.ShapeDtypeStruct((_E * _C + 8, _D), jnp.float32),
            jax.ShapeDtypeStruct((_E * _C + 8, 128), jnp.float32),
        ),
        mesh=plsc.VectorSubcoreMesh(core_axis_name="c", subcore_axis_name="s"),
        scratch_types=[
            pltpu.VMEM((_TPW,), jnp.int32),
            pltpu.VMEM((_TPW, _D), jnp.float32),
            pltpu.VMEM((_TPW, 128), jnp.float32),
            pltpu.SemaphoreType.DMA,
            pltpu.SemaphoreType.DMA,
        ],
    )(_dispatch_body)


def _ffn_body(x_ref, w1_ref, w2_ref, ss_ref, y_ref):
    e = pl.program_id(0)

    @pl.when(e < _E)
    def _():
        x = x_ref[...].astype(jnp.bfloat16)
        h = jnp.maximum(
            lax.dot_general(x, w1_ref[0].astype(jnp.bfloat16),
                            (((1,), (0,)), ((), ())),
                            preferred_element_type=jnp.float32), 0.0)
        y = lax.dot_general(h.astype(jnp.bfloat16),
                            w2_ref[0].astype(jnp.bfloat16),
                            (((1,), (0,)), ((), ())),
                            preferred_element_type=jnp.float32)
        y_ref[...] = y * ss_ref[:, 0:1]

    @pl.when(e == _E)
    def _():
        y_ref[...] = jnp.zeros((_C, _D), jnp.float32)


def _clampe(e):
    return jnp.minimum(e, _E - 1)


_ffn_call = pl.pallas_call(
    _ffn_body,
    grid=(_E + 1,),
    in_specs=[
        pl.BlockSpec((_C, _D), lambda e: (_clampe(e), 0)),
        pl.BlockSpec((1, _D, _F), lambda e: (_clampe(e), 0, 0)),
        pl.BlockSpec((1, _F, _D), lambda e: (_clampe(e), 0, 0)),
        pl.BlockSpec((_C, 128), lambda e: (_clampe(e), 0)),
    ],
    out_specs=pl.BlockSpec((_C, _D), lambda e: (e, 0)),
    out_shape=jax.ShapeDtypeStruct(((_E + 1) * _C, _D), jnp.float32),
    compiler_params=pltpu.CompilerParams(
        dimension_semantics=("arbitrary",)),
)


def _combine_body(y_hbm, gidx_hbm, ygath_hbm, idx_v, rows_v, sem):
    wid = lax.axis_index("s") * _NC + lax.axis_index("c")
    base = wid * _TPW
    pltpu.sync_copy(gidx_hbm.at[pl.ds(base, _TPW)], idx_v)
    pltpu.async_copy(y_hbm.at[idx_v], rows_v, sem).wait()
    pltpu.sync_copy(rows_v, ygath_hbm.at[pl.ds(base, _TPW)])


@functools.cache
def _combine_call():
    return functools.partial(
        pl.kernel,
        out_type=jax.ShapeDtypeStruct((_T, _D), jnp.float32),
        mesh=plsc.VectorSubcoreMesh(core_axis_name="c", subcore_axis_name="s"),
        scratch_types=[
            pltpu.VMEM((_TPW,), jnp.int32),
            pltpu.VMEM((_TPW, _D), jnp.float32),
            pltpu.SemaphoreType.DMA,
        ],
    )(_combine_body)


def kernel(hidden_states, w_router, w1, w2):
    B, S, D = hidden_states.shape
    flat = hidden_states.reshape(B * S, D)

    lt, eidx, sidx, gidx, gscale = _router_call(flat, w_router.T)
    ebuf, sbuf = _dispatch_call()(flat, sidx, gscale)
    y = _ffn_call(ebuf, w1, w2, sbuf)
    out = _combine_call()(y, gidx)

    return (out.reshape(B, S, D),
            (lt.T.reshape(B, S, _E), eidx.reshape(B, S)))


# speedup vs baseline: 1.3854x; 1.0109x over previous
import functools

import jax
import jax.numpy as jnp
from jax import lax
from jax.experimental import pallas as pl
from jax.experimental.pallas import tpu as pltpu
from jax.experimental.pallas import tpu_sc as plsc

_T = 2048
_D = 768
_E = 8
_F = 3072
_C = 512

_NC = 2
_NS = 16
_NW = _NC * _NS
_TPW = _T // _NW

_FB = 3072
_NFB = _F // _FB


def _router_body(x_ref, wrt_ref, lt_ref, eidx_ref, sidx_ref,
                 gidx_ref, gscale_ref):
    x = x_ref[...]
    wrt = wrt_ref[...]
    logits = lax.dot_general(x, wrt, (((1,), (1,)), ((), ())),
                             preferred_element_type=jnp.float32)
    lt = logits.T
    lt_ref[...] = lt

    m = jnp.max(lt, axis=0, keepdims=True)
    iota_e = lax.broadcasted_iota(jnp.int32, (_E, _T), 0)
    eidx = jnp.min(jnp.where(lt == m, iota_e, _E), axis=0,
                   keepdims=True)
    eidx_ref[...] = eidx.reshape(_T)

    p = 1.0 / jnp.sum(jnp.exp(lt - m), axis=0, keepdims=True)

    onehot = (iota_e == eidx).astype(jnp.float32)
    cum = onehot
    k = 1
    while k < _T:
        shifted = jnp.concatenate(
            [jnp.zeros((_E, k), jnp.float32), cum[:, :_T - k]], axis=1)
        cum = cum + shifted
        k *= 2
    loc = jnp.sum((cum - 1.0) * onehot, axis=0, keepdims=True)
    kept = loc < float(_C)
    loc_i = loc.astype(jnp.int32)
    slot = eidx * _C + loc_i

    sidx_ref[...] = jnp.where(kept, slot, _E * _C).reshape(_T)
    gidx_ref[...] = jnp.where(kept, slot, _E * _C).reshape(_T)
    gsc = jnp.where(kept, p, 0.0).reshape(_T, 1)
    gscale_ref[...] = jnp.broadcast_to(gsc, (_T, 128))


_router_call = pl.pallas_call(
    _router_body,
    out_shape=(
        jax.ShapeDtypeStruct((_E, _T), jnp.float32),
        jax.ShapeDtypeStruct((_T,), jnp.int32),
        jax.ShapeDtypeStruct((_T,), jnp.int32),
        jax.ShapeDtypeStruct((_T,), jnp.int32),
        jax.ShapeDtypeStruct((_T, 128), jnp.float32),
    ),
)


def _dispatch_body(flat_hbm, sidx_hbm, gs_hbm, ebuf_hbm, sbuf_hbm,
                   idx_v, rows_v, gs_v, sem, sem2, sem3):
    wid = lax.axis_index("s") * _NC + lax.axis_index("c")
    base = wid * _TPW
    ld1 = pltpu.async_copy(sidx_hbm.at[pl.ds(base, _TPW)], idx_v, sem)
    ld2 = pltpu.async_copy(flat_hbm.at[pl.ds(base, _TPW)], rows_v, sem2)
    ld3 = pltpu.async_copy(gs_hbm.at[pl.ds(base, _TPW)], gs_v, sem3)
    ld1.wait()
    ld2.wait()
    ld3.wait()
    cp1 = pltpu.async_copy(rows_v, ebuf_hbm.at[idx_v], sem)
    cp2 = pltpu.async_copy(gs_v, sbuf_hbm.at[idx_v], sem2)
    cp1.wait()
    cp2.wait()


@functools.cache
def _dispatch_call():
    return functools.partial(
        pl.kernel,
        out_type=(
            jax.ShapeDtypeStruct((_E * _C + 8, _D), jnp.float32),
            jax.ShapeDtypeStruct((_E * _C + 8, 128), jnp.float32),
        ),
        mesh=plsc.VectorSubcoreMesh(core_axis_name="c", subcore_axis_name="s"),
        scratch_types=[
            pltpu.VMEM((_TPW,), jnp.int32),
            pltpu.VMEM((_TPW, _D), jnp.float32),
            pltpu.VMEM((_TPW, 128), jnp.float32),
            pltpu.SemaphoreType.DMA,
            pltpu.SemaphoreType.DMA,
            pltpu.SemaphoreType.DMA,
        ],
    )(_dispatch_body)


def _ffn_body(x_ref, w1_ref, w2_ref, ss_ref, y_ref):
    e = pl.program_id(0)

    @pl.when(e < _E)
    def _():
        x = x_ref[...].astype(jnp.bfloat16)
        h = jnp.maximum(
            lax.dot_general(x, w1_ref[0].astype(jnp.bfloat16),
                            (((1,), (0,)), ((), ())),
                            preferred_element_type=jnp.float32), 0.0)
        y = lax.dot_general(h.astype(jnp.bfloat16),
                            w2_ref[0].astype(jnp.bfloat16),
                            (((1,), (0,)), ((), ())),
                            preferred_element_type=jnp.float32)
        y_ref[...] = y * ss_ref[:, 0:1]

    @pl.when(e == _E)
    def _():
        y_ref[...] = jnp.zeros((_C, _D), jnp.float32)


def _clampe(e):
    return jnp.minimum(e, _E - 1)


_ffn_call = pl.pallas_call(
    _ffn_body,
    grid=(_E + 1,),
    in_specs=[
        pl.BlockSpec((_C, _D), lambda e: (_clampe(e), 0)),
        pl.BlockSpec((1, _D, _F), lambda e: (_clampe(e), 0, 0)),
        pl.BlockSpec((1, _F, _D), lambda e: (_clampe(e), 0, 0)),
        pl.BlockSpec((_C, 128), lambda e: (_clampe(e), 0)),
    ],
    out_specs=pl.BlockSpec((_C, _D), lambda e: (e, 0)),
    out_shape=jax.ShapeDtypeStruct(((_E + 1) * _C, _D), jnp.float32),
    compiler_params=pltpu.CompilerParams(
        dimension_semantics=("arbitrary",)),
)


def _combine_body(y_hbm, gidx_hbm, ygath_hbm, idx_v, rows_v, sem, sem2):
    wid = lax.axis_index("s") * _NC + lax.axis_index("c")
    base = wid * _TPW
    half = _TPW // 2
    pltpu.sync_copy(gidx_hbm.at[pl.ds(base, _TPW)], idx_v)
    g0 = pltpu.async_copy(y_hbm.at[idx_v.at[pl.ds(0, half)]],
                          rows_v.at[pl.ds(0, half)], sem)
    g1 = pltpu.async_copy(y_hbm.at[idx_v.at[pl.ds(half, half)]],
                          rows_v.at[pl.ds(half, half)], sem2)
    g0.wait()
    w0 = pltpu.async_copy(rows_v.at[pl.ds(0, half)],
                          ygath_hbm.at[pl.ds(base, half)], sem)
    g1.wait()
    w1 = pltpu.async_copy(rows_v.at[pl.ds(half, half)],
                          ygath_hbm.at[pl.ds(base + half, half)], sem2)
    w0.wait()
    w1.wait()


@functools.cache
def _combine_call():
    return functools.partial(
        pl.kernel,
        out_type=jax.ShapeDtypeStruct((_T, _D), jnp.float32),
        mesh=plsc.VectorSubcoreMesh(core_axis_name="c", subcore_axis_name="s"),
        scratch_types=[
            pltpu.VMEM((_TPW,), jnp.int32),
            pltpu.VMEM((_TPW, _D), jnp.float32),
            pltpu.SemaphoreType.DMA,
            pltpu.SemaphoreType.DMA,
        ],
    )(_combine_body)


def kernel(hidden_states, w_router, w1, w2):
    B, S, D = hidden_states.shape
    flat = hidden_states.reshape(B * S, D)

    lt, eidx, sidx, gidx, gscale = _router_call(flat, w_router.T)
    ebuf, sbuf = _dispatch_call()(flat, sidx, gscale)
    y = _ffn_call(ebuf, w1, w2, sbuf)
    out = _combine_call()(y, gidx)

    return (out.reshape(B, S, D),
            (lt.T.reshape(B, S, _E), eidx.reshape(B, S)))
